# baseline (speedup 1.0000x reference)
import functools

import jax
import jax.numpy as jnp
from jax import lax
from jax.experimental import pallas as pl
from jax.experimental.pallas import tpu as pltpu
from jax.experimental.pallas import tpu_sc as plsc

B = 1024
S = 200
D = 64
N = B * S
NC = 2
NS = 16
NW = NC * NS
CHUNK = 128
PER_W = N // NW
CHUNKS_W = PER_W // CHUNK

_mesh = plsc.VectorSubcoreMesh(core_axis_name="c", subcore_axis_name="s")


@functools.partial(
    pl.kernel,
    mesh=_mesh,
    compiler_params=pltpu.CompilerParams(use_tc_tiling_on_sc=False),
    out_type=jax.ShapeDtypeStruct((N, D), jnp.float32),
    scratch_types=[
        pltpu.VMEM((CHUNKS_W, CHUNK), jnp.int32),
        pltpu.VMEM((CHUNK, D), jnp.float32),
        pltpu.SemaphoreType.DMA,
    ],
)
def _gather(idx_hbm, table_hbm, out_hbm, idx_v, rows_v, sem):
    wid = lax.axis_index("s") * NC + lax.axis_index("c")
    chunk0 = wid * CHUNKS_W
    pltpu.sync_copy(idx_hbm.at[wid], idx_v)

    def body(j, carry):
        pltpu.async_copy(table_hbm.at[idx_v.at[j]], rows_v, sem).wait()
        pltpu.sync_copy(rows_v, out_hbm.at[pl.ds((chunk0 + j) * CHUNK, CHUNK)])
        return carry

    lax.fori_loop(0, CHUNKS_W, body, 0)


def _mask_body(x_ref, o_ref):
    o_ref[...] = (x_ref[...] != 0).astype(jnp.float32)


_mask_call = pl.pallas_call(
    _mask_body,
    out_shape=jax.ShapeDtypeStruct((B, S), jnp.float32),
)


def kernel(x, weight):
    xt = jnp.transpose(x).reshape(NW, CHUNKS_W, CHUNK).astype(jnp.int32)
    out = _gather(xt, weight)
    mask = _mask_call(x)
    return out.reshape(S, B, D), mask


# speedup vs baseline: 1.5518x; 1.5518x over previous
import functools

import jax
import jax.numpy as jnp
from jax import lax
from jax.experimental import pallas as pl
from jax.experimental.pallas import tpu as pltpu
from jax.experimental.pallas import tpu_sc as plsc

B = 1024
S = 200
D = 64
N = B * S
NC = 2
NS = 16
NW = NC * NS
CHUNK = 128
PER_W = N // NW
CHUNKS_W = PER_W // CHUNK
K = 5
GROUPS = CHUNKS_W // K
ROWS_G = K * CHUNK

_mesh = plsc.VectorSubcoreMesh(core_axis_name="c", subcore_axis_name="s")


@functools.partial(
    pl.kernel,
    mesh=_mesh,
    compiler_params=pltpu.CompilerParams(use_tc_tiling_on_sc=False),
    out_type=jax.ShapeDtypeStruct((N, D), jnp.float32),
    scratch_types=[
        pltpu.VMEM((CHUNKS_W, CHUNK), jnp.int32),
        pltpu.VMEM((2, ROWS_G, D), jnp.float32),
        pltpu.SemaphoreType.DMA,
        pltpu.SemaphoreType.DMA,
        pltpu.SemaphoreType.DMA,
        pltpu.SemaphoreType.DMA,
    ],
)
def _gather(idx_hbm, table_hbm, out_hbm, idx_v, bufs, g0, g1, w0, w1):
    wid = lax.axis_index("s") * NC + lax.axis_index("c")
    chunk0 = wid * CHUNKS_W
    gsem = (g0, g1)
    wsem = (w0, w1)
    pltpu.sync_copy(idx_hbm.at[wid], idx_v)

    def start_gather(g, b):
        for k in range(K):
            pltpu.async_copy(
                table_hbm.at[idx_v.at[g * K + k]],
                bufs.at[b, pl.ds(k * CHUNK, CHUNK)],
                gsem[b],
            )

    def drain_gather(g, b):
        for k in range(K):
            pltpu.make_async_copy(
                table_hbm.at[idx_v.at[g * K + k]],
                bufs.at[b, pl.ds(k * CHUNK, CHUNK)],
                gsem[b],
            ).wait()

    def writeback(g, b):
        return pltpu.make_async_copy(
            bufs.at[b],
            out_hbm.at[pl.ds((chunk0 + g * K) * CHUNK, ROWS_G)],
            wsem[b],
        )

    start_gather(0, 0)
    start_gather(1, 1)

    def body(gg, carry):
        for b in range(2):
            g = 2 * gg + b
            drain_gather(g, b)
            writeback(g, b).start()

            @pl.when(g + 2 < GROUPS)
            def _():
                writeback(g, b).wait()
                start_gather(g + 2, b)

        return carry

    lax.fori_loop(0, GROUPS // 2, body, 0)
    writeback(GROUPS - 2, 0).wait()
    writeback(GROUPS - 1, 1).wait()


def _mask_body(x_ref, o_ref):
    o_ref[...] = (x_ref[...] != 0).astype(jnp.float32)


_mask_call = pl.pallas_call(
    _mask_body,
    out_shape=jax.ShapeDtypeStruct((B, S), jnp.float32),
)


def kernel(x, weight):
    xt = jnp.transpose(x).reshape(NW, CHUNKS_W, CHUNK).astype(jnp.int32)
    out = _gather(xt, weight)
    mask = _mask_call(x)
    return out.reshape(S, B, D), mask
